# baseline (speedup 1.0000x reference)
import functools

import jax
import jax.numpy as jnp
from jax import lax
from jax.experimental import pallas as pl
from jax.experimental.pallas import tpu as pltpu
from jax.experimental.pallas import tpu_sc as plsc

H = 64
WPAD = 128
NC = 2
NS = 16
NW = NC * NS
LANES = 16
NBUF = 2
NOBUF = 2


def _body(xf_hbm, tv_hbm, out_hbm, idx_v, gbufs, obufs, gsems, osems):
    n_total = xf_hbm.shape[0]
    L = out_hbm.shape[1]
    n_b = n_total // L // NW
    wid = lax.axis_index("s") * NC + lax.axis_index("c")
    base = wid * n_b * L

    pltpu.sync_copy(
        xf_hbm.at[pl.ds(base, n_b * L)], idx_v.at[pl.ds(0, n_b * L)])

    splits = [(0, 128), (128, L - 128)]

    def start_gather(u, gb):
        off = u * L
        for s, n in splits:
            pltpu.async_copy(
                tv_hbm.at[idx_v.at[pl.ds(off + s, n)]],
                gbufs[gb].at[pl.ds(s, n)], gsems[gb])

    def wait_gather(gb):
        for s, n in splits:
            pltpu.make_async_copy(
                tv_hbm.at[pl.ds(0, n)], gbufs[gb].at[pl.ds(s, n)],
                gsems[gb]).wait()

    def wait_out(ob):
        pltpu.make_async_copy(
            obufs[ob], out_hbm.at[0], osems[ob]).wait()

    def compute(u, gb, ob):
        gbuf, obuf = gbufs[gb], obufs[ob]
        uoff = u * L
        for g in range(pl.cdiv(L, LANES)):
            n = min(LANES, L - g * LANES)
            idx = idx_v[pl.ds(uoff + g * LANES, LANES)]
            m = jnp.where(idx == 0, jnp.float32(0.0), jnp.float32(8.0))
            for j in range(n):
                mj = m[j]
                row = g * LANES + j
                for c in range(H // LANES):
                    sl = pl.ds(c * LANES, LANES)
                    obuf[row, sl] = gbuf[row, sl] * mj

    for gb in range(NBUF):
        start_gather(gb, gb)

    def outer_body(k, _):
        for t in range(NBUF):
            u = k * NBUF + t
            ob = t % NOBUF
            wait_gather(t)

            @pl.when(u >= NOBUF)
            def _():
                wait_out(ob)

            compute(u, t, ob)
            pltpu.async_copy(obufs[ob], out_hbm.at[wid * n_b + u], osems[ob])

            @pl.when(u + NBUF < n_b)
            def _():
                start_gather(u + NBUF, t)
        return ()

    lax.fori_loop(0, n_b // NBUF, outer_body, ())
    for ob in range(NOBUF):
        wait_out(ob)


def kernel(x, shared_weights):
    b, l = x.shape
    v = shared_weights.shape[0]
    xf = x.reshape(b * l).astype(jnp.int32)
    tv = jnp.pad(shared_weights, ((0, 0), (0, WPAD - H)))
    n_b = b // NW
    call = functools.partial(
        pl.kernel,
        mesh=plsc.VectorSubcoreMesh(core_axis_name="c", subcore_axis_name="s"),
        out_type=jax.ShapeDtypeStruct((b, l, H), jnp.float32),
        scratch_types=[
            pltpu.VMEM((n_b * l + LANES,), jnp.int32),
            [pltpu.VMEM((l, WPAD), jnp.float32) for _ in range(NBUF)],
            [pltpu.VMEM((l, H), jnp.float32) for _ in range(NOBUF)],
            [pltpu.SemaphoreType.DMA for _ in range(NBUF)],
            [pltpu.SemaphoreType.DMA for _ in range(NOBUF)],
        ],
        compiler_params=pltpu.CompilerParams(
            use_tc_tiling_on_sc=False, skip_device_barrier=True),
    )(_body)
    return call(xf, tv)


# speedup vs baseline: 1.1443x; 1.1443x over previous
import functools

import jax
import jax.numpy as jnp
from jax import lax
from jax.experimental import pallas as pl
from jax.experimental.pallas import tpu as pltpu
from jax.experimental.pallas import tpu_sc as plsc

H = 64
WPAD = 128
NC = 2
NS = 16
NW = NC * NS
LANES = 16
NBUF = 2
NOBUF = 2


def _body(xf_hbm, tv_hbm, out_hbm, idx_v, gbufs, obufs, gsems, osems):
    n_total = xf_hbm.shape[0]
    L = out_hbm.shape[1]
    n_b = n_total // L // NW
    wid = lax.axis_index("s") * NC + lax.axis_index("c")
    base = wid * n_b * L

    pltpu.sync_copy(
        xf_hbm.at[pl.ds(base, n_b * L)], idx_v.at[pl.ds(0, n_b * L)])

    splits = [(0, 128), (128, L - 128)]

    def start_gather(u, gb):
        off = u * L
        for s, n in splits:
            pltpu.async_copy(
                tv_hbm.at[idx_v.at[pl.ds(off + s, n)]],
                gbufs[gb].at[pl.ds(s, n)], gsems[gb])

    def wait_gather(gb):
        for s, n in splits:
            pltpu.make_async_copy(
                tv_hbm.at[pl.ds(0, n)], gbufs[gb].at[pl.ds(s, n)],
                gsems[gb]).wait()

    def wait_out(ob):
        pltpu.make_async_copy(
            obufs[ob], out_hbm.at[0], osems[ob]).wait()

    def compute(u, gb, ob):
        gbuf, obuf = gbufs[gb], obufs[ob]
        uoff = u * L
        for g in range(pl.cdiv(L, LANES)):
            n = min(LANES, L - g * LANES)
            idx = idx_v[pl.ds(uoff + g * LANES, LANES)]
            m = jnp.where(idx == 0, jnp.float32(0.0), jnp.float32(8.0))
            for j in range(n):
                mj = m[j]
                row = g * LANES + j
                for c in range(H // LANES):
                    sl = pl.ds(c * LANES, LANES)
                    obuf[row, sl] = gbuf[row, sl] * mj

    for gb in range(NBUF):
        start_gather(gb, gb)

    def outer_body(k, _):
        for t in range(NBUF):
            u = k * NBUF + t
            ob = t % NOBUF
            wait_gather(t)

            @pl.when(u >= NOBUF)
            def _():
                wait_out(ob)

            compute(u, t, ob)
            pltpu.async_copy(obufs[ob], out_hbm.at[wid * n_b + u], osems[ob])

            @pl.when(u + NBUF < n_b)
            def _():
                start_gather(u + NBUF, t)
        return ()

    lax.fori_loop(0, n_b // NBUF, outer_body, ())
    for ob in range(NOBUF):
        wait_out(ob)


def kernel(x, shared_weights):
    b, l = x.shape
    v = shared_weights.shape[0]
    xf = x.reshape(b * l).astype(jnp.int32)
    tv = jnp.concatenate(
        [shared_weights, jnp.zeros((v, WPAD - H), jnp.float32)], axis=1)
    n_b = b // NW
    call = functools.partial(
        pl.kernel,
        mesh=plsc.VectorSubcoreMesh(core_axis_name="c", subcore_axis_name="s"),
        out_type=jax.ShapeDtypeStruct((b, l, H), jnp.float32),
        scratch_types=[
            pltpu.VMEM((n_b * l + LANES,), jnp.int32),
            [pltpu.VMEM((l, WPAD), jnp.float32) for _ in range(NBUF)],
            [pltpu.VMEM((l, H), jnp.float32) for _ in range(NOBUF)],
            [pltpu.SemaphoreType.DMA for _ in range(NBUF)],
            [pltpu.SemaphoreType.DMA for _ in range(NOBUF)],
        ],
        compiler_params=pltpu.CompilerParams(use_tc_tiling_on_sc=True),
    )(_body)
    return call(xf, tv)


# speedup vs baseline: 1.2577x; 1.0991x over previous
import functools

import jax
import jax.numpy as jnp
from jax import lax
from jax.experimental import pallas as pl
from jax.experimental.pallas import tpu as pltpu
from jax.experimental.pallas import tpu_sc as plsc

H = 64
WPAD = 128
NC = 2
NS = 16
NW = NC * NS
LANES = 16
NBUF = 2
NOBUF = 2


def _body(xf_hbm, tv_hbm, out_hbm, idx_v, gbufs, obufs, gsems, osems):
    n_total = xf_hbm.shape[0]
    L = out_hbm.shape[1]
    n_b = n_total // L // NW
    wid = lax.axis_index("s") * NC + lax.axis_index("c")
    base = wid * n_b * L

    pltpu.sync_copy(
        xf_hbm.at[pl.ds(base, n_b * L)], idx_v.at[pl.ds(0, n_b * L)])

    splits = [(0, 128), (128, L - 128)]

    def start_gather(u, gb):
        off = u * L
        for s, n in splits:
            pltpu.async_copy(
                tv_hbm.at[idx_v.at[pl.ds(off + s, n)]],
                gbufs[gb].at[pl.ds(s, n)], gsems[gb])

    def wait_gather(gb):
        for s, n in splits:
            pltpu.make_async_copy(
                tv_hbm.at[pl.ds(0, n)], gbufs[gb].at[pl.ds(s, n)],
                gsems[gb]).wait()

    def wait_out(ob):
        pltpu.make_async_copy(
            obufs[ob], out_hbm.at[0], osems[ob]).wait()

    def compute(u, gb, ob):
        gbuf, obuf = gbufs[gb], obufs[ob]
        uoff = u * L
        for g in range(pl.cdiv(L, LANES)):
            n = min(LANES, L - g * LANES)
            idx = idx_v[pl.ds(uoff + g * LANES, LANES)]
            m = jnp.where(idx == 0, jnp.float32(0.0), jnp.float32(8.0))
            for j in range(n):
                mj = m[j]
                row = g * LANES + j
                for c in range(H // LANES):
                    sl = pl.ds(c * LANES, LANES)
                    obuf[row, sl] = gbuf[row, sl] * mj

    for gb in range(NBUF):
        start_gather(gb, gb)

    def outer_body(k, _):
        for t in range(NBUF):
            u = k * NBUF + t
            ob = t % NOBUF
            wait_gather(t)

            @pl.when(u >= NOBUF)
            def _():
                wait_out(ob)

            compute(u, t, ob)
            pltpu.async_copy(obufs[ob], out_hbm.at[wid * n_b + u], osems[ob])

            @pl.when(u + NBUF < n_b)
            def _():
                start_gather(u + NBUF, t)
        return ()

    lax.fori_loop(0, n_b // NBUF, outer_body, ())
    for ob in range(NOBUF):
        wait_out(ob)


def kernel(x, shared_weights):
    b, l = x.shape
    v = shared_weights.shape[0]
    xf = x.reshape(b * l).astype(jnp.int32)
    tv = jnp.concatenate(
        [shared_weights, jnp.zeros((v, WPAD - H), jnp.float32)], axis=1)
    n_b = b // NW
    call = functools.partial(
        pl.kernel,
        mesh=plsc.VectorSubcoreMesh(core_axis_name="c", subcore_axis_name="s"),
        out_type=jax.ShapeDtypeStruct((b, l, H), jnp.float32),
        scratch_types=[
            pltpu.VMEM((n_b * l + LANES,), jnp.int32),
            [pltpu.VMEM((l, WPAD), jnp.float32) for _ in range(NBUF)],
            [pltpu.VMEM((l, H), jnp.float32) for _ in range(NOBUF)],
            [pltpu.SemaphoreType.DMA for _ in range(NBUF)],
            [pltpu.SemaphoreType.DMA for _ in range(NOBUF)],
        ],
        compiler_params=pltpu.CompilerParams(use_tc_tiling_on_sc=True),
    )(_body)
    return lax.optimization_barrier(call(xf, tv))
